# baseline (speedup 1.0000x reference)
import functools

import jax
import jax.numpy as jnp
from jax import lax
from jax.experimental import pallas as pl
from jax.experimental.pallas import tpu as pltpu
from jax.experimental.pallas import tpu_sc as plsc

N_NODES = 10000
N_EDGES = 320000
D_FEAT = 128
D_EDGE = 16
D_RADIAL = 8
D_HIDDEN = 64
INV_AVG = 1.0 / 32.0

_NC = 2
_NS = 16
_NW = _NC * _NS
_C = 40
_N_PAD = 10240
_ROWS_PER_TILE = _N_PAD // _NS
_LANES = D_FEAT // 16
_EB = 6400

_E_A = 96000
_E_B = N_EDGES - _E_A

_NB = 4


def _pack_bf16(lo, hi):
    li = lax.bitcast_convert_type(lo, jnp.int32) + jnp.int32(0x8000)
    hi_i = lax.bitcast_convert_type(hi, jnp.int32) + jnp.int32(0x8000)
    return lax.bitwise_or(lax.shift_right_logical(li, 16),
                          lax.bitwise_and(hi_i, jnp.int32(-65536)))


def _xup_body(node_ref, wup_ref, out_ref):
    out_ref[...] = jnp.dot(node_ref[...], wup_ref[...],
                           preferred_element_type=jnp.float32)


_DN0 = (((0,), (0,)), ((), ()))


def _edge_body(eft_ref, radt_ref, w1_ref, w2_ref, wtp_ref, out_ref):
    h_t = lax.dot_general(w1_ref[...], radt_ref[...], _DN0,
                          preferred_element_type=jnp.float32)
    h_t = h_t * jax.nn.sigmoid(h_t)
    tpw_t = lax.dot_general(w2_ref[...], h_t, _DN0,
                            preferred_element_type=jnp.float32)
    u_t = eft_ref[...] * tpw_t
    m = lax.dot_general(u_t, wtp_ref[...], _DN0,
                        preferred_element_type=jnp.float32)
    out_ref[...] = _pack_bf16(m[:, :64], m[:, 64:])


def _down_body(p_ref, q_ref, wd_ref, out_ref):
    p = p_ref[...]
    q = q_ref[...]
    agg = ((p[0] + p[1]) + (q[0] + q[1])) * INV_AVG
    out_ref[...] = jnp.dot(agg, wd_ref[...], preferred_element_type=jnp.float32)


def _make_sc_body(base_e, ew):
    nchunk = ew // _C

    def _sc_body(x_hbm, mix_hbm, send_hbm, recv_hbm, out_hbm, acc, *scratch):
        sidx = scratch[0:_NB]
        ridx = scratch[_NB:2 * _NB]
        rows = scratch[2 * _NB:3 * _NB]
        mixv = scratch[3 * _NB:4 * _NB]
        sem_si = scratch[4 * _NB:5 * _NB]
        sem_ri = scratch[5 * _NB:6 * _NB]
        sem_mx = scratch[6 * _NB:7 * _NB]
        sem_g = scratch[7 * _NB:8 * _NB]
        sem_sc = scratch[8 * _NB:9 * _NB]

        c = lax.axis_index("c")
        s = lax.axis_index("s")
        wid = c * _NS + s
        mbase0 = wid * ew
        ebase0 = base_e + mbase0

        def _zero_row(r, carry):
            for j in range(_LANES):
                rows[0][r, pl.ds(j * 16, 16)] = jnp.zeros((16,), jnp.float32)
            return carry
        lax.fori_loop(0, _C, _zero_row, None)
        my_row0 = s * _ROWS_PER_TILE
        for t in range(_ROWS_PER_TILE // _C):
            pltpu.async_copy(rows[0], acc.at[pl.ds(my_row0 + t * _C, _C)], sem_g[0])
        for t in range(_ROWS_PER_TILE // _C):
            pltpu.make_async_copy(rows[0], acc.at[pl.ds(my_row0, _C)], sem_g[0]).wait()
        plsc.subcore_barrier()

        def _start_in(j, b):
            eb_ = pl.multiple_of(ebase0 + j * _C, 8)
            mb_ = pl.multiple_of(mbase0 + j * _C, 8)
            pltpu.async_copy(send_hbm.at[pl.ds(eb_, _C)], sidx[b], sem_si[b])
            pltpu.async_copy(recv_hbm.at[pl.ds(eb_, _C)], ridx[b], sem_ri[b])
            pltpu.async_copy(mix_hbm.at[pl.ds(mb_, _C)], mixv[b], sem_mx[b])

        def _start_gather(b):
            pltpu.make_async_copy(send_hbm.at[pl.ds(0, _C)], sidx[b], sem_si[b]).wait()
            pltpu.async_copy(x_hbm.at[sidx[b]], rows[b], sem_g[b])

        def _drain_scatter(b):
            pltpu.make_async_copy(rows[b], acc.at[ridx[b]], sem_sc[b]).wait()

        _start_in(0, 0)
        _start_in(1, 1)
        _start_gather(0)

        def _group(i, carry):
            k0 = i * _NB
            for b in range(_NB):
                k = k0 + b

                b2 = (b + 2) % _NB
                @pl.when(k + 2 <= nchunk - 1)
                def _():
                    @pl.when(k - 2 >= 0)
                    def _():
                        _drain_scatter(b2)
                    _start_in(k + 2, b2)

                b1 = (b + 1) % _NB
                @pl.when(k + 1 <= nchunk - 1)
                def _():
                    _start_gather(b1)

                @pl.when(k <= nchunk - 1)
                def _():
                    pltpu.make_async_copy(
                        mix_hbm.at[pl.ds(0, _C)], mixv[b], sem_mx[b]).wait()
                    pltpu.make_async_copy(x_hbm.at[sidx[b]], rows[b], sem_g[b]).wait()

                    mask = jnp.int32(-65536)
                    bcc = lax.bitcast_convert_type

                    def _mul_row(r2, carry2):
                        for u in range(2):
                            r = r2 * 2 + u
                            for j in range(4):
                                sl = pl.ds(j * 16, 16)
                                sh = pl.ds(64 + j * 16, 16)
                                wm = mixv[b][r, sl]
                                lo_m = bcc(lax.shift_left(wm, 16), jnp.float32)
                                hi_m = bcc(lax.bitwise_and(wm, mask), jnp.float32)
                                rows[b][r, sl] = rows[b][r, sl] * lo_m
                                rows[b][r, sh] = rows[b][r, sh] * hi_m
                        return carry2
                    lax.fori_loop(0, _C // 2, _mul_row, None)

                    pltpu.make_async_copy(
                        recv_hbm.at[pl.ds(0, _C)], ridx[b], sem_ri[b]).wait()
                    pltpu.async_copy(rows[b], acc.at[ridx[b]], sem_sc[b], add=True)
            return carry
        n_groups = (nchunk + _NB - 1) // _NB
        lax.fori_loop(0, n_groups, _group, None)

        for b in range(_NB):
            _drain_scatter(b)

        plsc.subcore_barrier()
        for t in range(_ROWS_PER_TILE // _C):
            b = t % 2
            r0 = my_row0 + t * _C
            if t >= 2:
                pltpu.make_async_copy(
                    rows[b], out_hbm.at[c, pl.ds(my_row0, _C)], sem_sc[b]).wait()
            pltpu.sync_copy(acc.at[pl.ds(r0, _C)], rows[b])
            pltpu.async_copy(rows[b], out_hbm.at[c, pl.ds(r0, _C)], sem_sc[b])
        for b in range(2):
            pltpu.make_async_copy(
                rows[b], out_hbm.at[c, pl.ds(my_row0, _C)], sem_sc[b]).wait()

    return _sc_body


@functools.lru_cache(maxsize=None)
def _sc_call(base_e, ew):
    return pl.kernel(
        _make_sc_body(base_e, ew),
        out_type=jax.ShapeDtypeStruct((_NC, _N_PAD, D_FEAT), jnp.float32),
        mesh=plsc.VectorSubcoreMesh(core_axis_name="c", subcore_axis_name="s"),
        scratch_types=(
            [pltpu.VMEM_SHARED((_N_PAD, D_FEAT), jnp.float32)]
            + [pltpu.VMEM((_C,), jnp.int32)] * _NB
            + [pltpu.VMEM((_C,), jnp.int32)] * _NB
            + [pltpu.VMEM((_C, D_FEAT), jnp.float32)] * _NB
            + [pltpu.VMEM((_C, 64), jnp.int32)] * _NB
            + [pltpu.SemaphoreType.DMA] * (5 * _NB)
        ),
    )


def kernel(node_features, edge_features, radial_embedding, senders, receivers,
           W_up, W_mlp1, W_mlp2, W_tp, W_down):
    f32 = jnp.float32
    nb = 1000

    x = pl.pallas_call(
        _xup_body,
        grid=(N_NODES // nb,),
        in_specs=[
            pl.BlockSpec((nb, D_FEAT), lambda i: (i, 0)),
            pl.BlockSpec((D_FEAT, D_FEAT), lambda i: (0, 0)),
        ],
        out_specs=pl.BlockSpec((nb, D_FEAT), lambda i: (i, 0)),
        out_shape=jax.ShapeDtypeStruct((N_NODES, D_FEAT), f32),
    )(node_features, W_up)

    eft = edge_features.T
    radt = radial_embedding.T

    def mix_slice(base_e, n_e, wtp):
        off = base_e // _EB
        return pl.pallas_call(
            _edge_body,
            grid=(n_e // _EB,),
            in_specs=[
                pl.BlockSpec((D_EDGE, _EB), lambda i: (0, i + off)),
                pl.BlockSpec((D_RADIAL, _EB), lambda i: (0, i + off)),
                pl.BlockSpec((D_RADIAL, D_HIDDEN), lambda i: (0, 0)),
                pl.BlockSpec((D_HIDDEN, D_EDGE), lambda i: (0, 0)),
                pl.BlockSpec((D_EDGE, D_FEAT), lambda i: (0, 0)),
            ],
            out_specs=pl.BlockSpec((_EB, 64), lambda i: (i, 0)),
            out_shape=jax.ShapeDtypeStruct((n_e, 64), jnp.int32),
        )(eft, radt, W_mlp1, W_mlp2, wtp)

    mix_a = mix_slice(0, _E_A, W_tp)
    dep = mix_a[0, 0].astype(f32) * 0.0
    mix_b = mix_slice(_E_A, _E_B, W_tp + dep)
    partials_a = _sc_call(0, _E_A // _NW)(x, mix_a, senders, receivers)
    partials_b = _sc_call(_E_A, _E_B // _NW)(x, mix_b, senders, receivers)

    out = pl.pallas_call(
        _down_body,
        grid=(N_NODES // nb,),
        in_specs=[
            pl.BlockSpec((_NC, nb, D_FEAT), lambda i: (0, i, 0)),
            pl.BlockSpec((_NC, nb, D_FEAT), lambda i: (0, i, 0)),
            pl.BlockSpec((D_FEAT, D_FEAT), lambda i: (0, 0)),
        ],
        out_specs=pl.BlockSpec((nb, D_FEAT), lambda i: (i, 0)),
        out_shape=jax.ShapeDtypeStruct((N_NODES, D_FEAT), f32),
    )(partials_a, partials_b, W_down)
    return out


# speedup vs baseline: 1.0036x; 1.0036x over previous
import functools

import jax
import jax.numpy as jnp
from jax import lax
from jax.experimental import pallas as pl
from jax.experimental.pallas import tpu as pltpu
from jax.experimental.pallas import tpu_sc as plsc

N_NODES = 10000
N_EDGES = 320000
D_FEAT = 128
D_EDGE = 16
D_RADIAL = 8
D_HIDDEN = 64
INV_AVG = 1.0 / 32.0

_NC = 2
_NS = 16
_NW = _NC * _NS
_C = 40
_N_PAD = 10240
_ROWS_PER_TILE = _N_PAD // _NS
_LANES = D_FEAT // 16
_EB = 6400

_E_A = 96000
_E_B = N_EDGES - _E_A

_NB = 4


def _pack_bf16(lo, hi):
    li = lax.bitcast_convert_type(lo, jnp.int32) + jnp.int32(0x8000)
    hi_i = lax.bitcast_convert_type(hi, jnp.int32) + jnp.int32(0x8000)
    return lax.bitwise_or(lax.shift_right_logical(li, 16),
                          lax.bitwise_and(hi_i, jnp.int32(-65536)))


def _xup_body(node_ref, wup_ref, out_ref):
    out_ref[...] = jnp.dot(node_ref[...], wup_ref[...],
                           preferred_element_type=jnp.float32)


_DN0 = (((0,), (0,)), ((), ()))


def _edge_body(eft_ref, radt_ref, w1_ref, w2_ref, wtp_ref, out_ref):
    h_t = lax.dot_general(w1_ref[...], radt_ref[...], _DN0,
                          preferred_element_type=jnp.float32)
    h_t = h_t * jax.nn.sigmoid(h_t)
    tpw_t = lax.dot_general(w2_ref[...], h_t, _DN0,
                            preferred_element_type=jnp.float32)
    u_t = eft_ref[...] * tpw_t
    m = lax.dot_general(u_t, wtp_ref[...], _DN0,
                        preferred_element_type=jnp.float32)
    out_ref[...] = _pack_bf16(m[:, :64], m[:, 64:])


def _down_body(p_ref, q_ref, wd_ref, out_ref):
    p = p_ref[...]
    q = q_ref[...]
    agg = ((p[0] + p[1]) + (q[0] + q[1])) * INV_AVG
    out_ref[...] = jnp.dot(agg, wd_ref[...], preferred_element_type=jnp.float32)


def _make_sc_body(base_e, ew):
    nchunk = ew // _C

    def _sc_body(x_hbm, mix_hbm, send_hbm, recv_hbm, out_hbm, acc, *scratch):
        sidx = scratch[0:_NB]
        ridx = scratch[_NB:2 * _NB]
        rows = scratch[2 * _NB:3 * _NB]
        mixv = scratch[3 * _NB:4 * _NB]
        sem_si = scratch[4 * _NB:5 * _NB]
        sem_ri = scratch[5 * _NB:6 * _NB]
        sem_mx = scratch[6 * _NB:7 * _NB]
        sem_g = scratch[7 * _NB:8 * _NB]
        sem_sc = scratch[8 * _NB:9 * _NB]

        c = lax.axis_index("c")
        s = lax.axis_index("s")
        wid = c * _NS + s
        mbase0 = wid * ew
        ebase0 = base_e + mbase0

        def _zero_row(r, carry):
            for j in range(_LANES):
                rows[0][r, pl.ds(j * 16, 16)] = jnp.zeros((16,), jnp.float32)
            return carry
        lax.fori_loop(0, _C, _zero_row, None)
        my_row0 = s * _ROWS_PER_TILE
        for t in range(_ROWS_PER_TILE // _C):
            pltpu.async_copy(rows[0], acc.at[pl.ds(my_row0 + t * _C, _C)], sem_g[0])
        for t in range(_ROWS_PER_TILE // _C):
            pltpu.make_async_copy(rows[0], acc.at[pl.ds(my_row0, _C)], sem_g[0]).wait()
        plsc.subcore_barrier()

        def _start_in(j, b):
            eb_ = pl.multiple_of(ebase0 + j * _C, 8)
            mb_ = pl.multiple_of(mbase0 + j * _C, 8)
            pltpu.async_copy(send_hbm.at[pl.ds(eb_, _C)], sidx[b], sem_si[b])
            pltpu.async_copy(recv_hbm.at[pl.ds(eb_, _C)], ridx[b], sem_ri[b])
            pltpu.async_copy(mix_hbm.at[pl.ds(mb_, _C)], mixv[b], sem_mx[b])

        def _start_gather(b):
            pltpu.make_async_copy(send_hbm.at[pl.ds(0, _C)], sidx[b], sem_si[b]).wait()
            pltpu.async_copy(x_hbm.at[sidx[b]], rows[b], sem_g[b])

        def _drain_scatter(b):
            pltpu.make_async_copy(rows[b], acc.at[ridx[b]], sem_sc[b]).wait()

        _start_in(0, 0)
        _start_in(1, 1)
        _start_gather(0)

        def _group(i, carry):
            k0 = i * _NB
            for b in range(_NB):
                k = k0 + b

                b2 = (b + 2) % _NB
                @pl.when(k + 2 <= nchunk - 1)
                def _():
                    @pl.when(k - 2 >= 0)
                    def _():
                        _drain_scatter(b2)
                    _start_in(k + 2, b2)

                b1 = (b + 1) % _NB
                @pl.when(k + 1 <= nchunk - 1)
                def _():
                    _start_gather(b1)

                @pl.when(k <= nchunk - 1)
                def _():
                    pltpu.make_async_copy(
                        mix_hbm.at[pl.ds(0, _C)], mixv[b], sem_mx[b]).wait()
                    pltpu.make_async_copy(x_hbm.at[sidx[b]], rows[b], sem_g[b]).wait()

                    mask = jnp.int32(-65536)
                    bcc = lax.bitcast_convert_type

                    def _mul_row(r, carry2):
                        for j in range(4):
                            sl = pl.ds(j * 16, 16)
                            sh = pl.ds(64 + j * 16, 16)
                            wm = mixv[b][r, sl]
                            lo_m = bcc(lax.shift_left(wm, 16), jnp.float32)
                            hi_m = bcc(lax.bitwise_and(wm, mask), jnp.float32)
                            rows[b][r, sl] = rows[b][r, sl] * lo_m
                            rows[b][r, sh] = rows[b][r, sh] * hi_m
                        return carry2
                    lax.fori_loop(0, _C, _mul_row, None)

                    pltpu.make_async_copy(
                        recv_hbm.at[pl.ds(0, _C)], ridx[b], sem_ri[b]).wait()
                    pltpu.async_copy(rows[b], acc.at[ridx[b]], sem_sc[b], add=True)
            return carry
        n_groups = (nchunk + _NB - 1) // _NB
        lax.fori_loop(0, n_groups, _group, None)

        for b in range(_NB):
            _drain_scatter(b)

        plsc.subcore_barrier()
        for t in range(_ROWS_PER_TILE // _C):
            b = t % 2
            r0 = my_row0 + t * _C
            if t >= 2:
                pltpu.make_async_copy(
                    rows[b], out_hbm.at[c, pl.ds(my_row0, _C)], sem_sc[b]).wait()
            pltpu.sync_copy(acc.at[pl.ds(r0, _C)], rows[b])
            pltpu.async_copy(rows[b], out_hbm.at[c, pl.ds(r0, _C)], sem_sc[b])
        for b in range(2):
            pltpu.make_async_copy(
                rows[b], out_hbm.at[c, pl.ds(my_row0, _C)], sem_sc[b]).wait()

    return _sc_body


@functools.lru_cache(maxsize=None)
def _sc_call(base_e, ew):
    return pl.kernel(
        _make_sc_body(base_e, ew),
        out_type=jax.ShapeDtypeStruct((_NC, _N_PAD, D_FEAT), jnp.float32),
        mesh=plsc.VectorSubcoreMesh(core_axis_name="c", subcore_axis_name="s"),
        scratch_types=(
            [pltpu.VMEM_SHARED((_N_PAD, D_FEAT), jnp.float32)]
            + [pltpu.VMEM((_C,), jnp.int32)] * _NB
            + [pltpu.VMEM((_C,), jnp.int32)] * _NB
            + [pltpu.VMEM((_C, D_FEAT), jnp.float32)] * _NB
            + [pltpu.VMEM((_C, 64), jnp.int32)] * _NB
            + [pltpu.SemaphoreType.DMA] * (5 * _NB)
        ),
    )


def kernel(node_features, edge_features, radial_embedding, senders, receivers,
           W_up, W_mlp1, W_mlp2, W_tp, W_down):
    f32 = jnp.float32
    nb = 1000

    x = pl.pallas_call(
        _xup_body,
        grid=(N_NODES // nb,),
        in_specs=[
            pl.BlockSpec((nb, D_FEAT), lambda i: (i, 0)),
            pl.BlockSpec((D_FEAT, D_FEAT), lambda i: (0, 0)),
        ],
        out_specs=pl.BlockSpec((nb, D_FEAT), lambda i: (i, 0)),
        out_shape=jax.ShapeDtypeStruct((N_NODES, D_FEAT), f32),
    )(node_features, W_up)

    eft = edge_features.T
    radt = radial_embedding.T

    def mix_slice(base_e, n_e, wtp):
        off = base_e // _EB
        return pl.pallas_call(
            _edge_body,
            grid=(n_e // _EB,),
            in_specs=[
                pl.BlockSpec((D_EDGE, _EB), lambda i: (0, i + off)),
                pl.BlockSpec((D_RADIAL, _EB), lambda i: (0, i + off)),
                pl.BlockSpec((D_RADIAL, D_HIDDEN), lambda i: (0, 0)),
                pl.BlockSpec((D_HIDDEN, D_EDGE), lambda i: (0, 0)),
                pl.BlockSpec((D_EDGE, D_FEAT), lambda i: (0, 0)),
            ],
            out_specs=pl.BlockSpec((_EB, 64), lambda i: (i, 0)),
            out_shape=jax.ShapeDtypeStruct((n_e, 64), jnp.int32),
        )(eft, radt, W_mlp1, W_mlp2, wtp)

    mix_a = mix_slice(0, _E_A, W_tp)
    dep = mix_a[0, 0].astype(f32) * 0.0
    mix_b = mix_slice(_E_A, _E_B, W_tp + dep)
    partials_a = _sc_call(0, _E_A // _NW)(x, mix_a, senders, receivers)
    partials_b = _sc_call(_E_A, _E_B // _NW)(x, mix_b, senders, receivers)

    out = pl.pallas_call(
        _down_body,
        grid=(N_NODES // nb,),
        in_specs=[
            pl.BlockSpec((_NC, nb, D_FEAT), lambda i: (0, i, 0)),
            pl.BlockSpec((_NC, nb, D_FEAT), lambda i: (0, i, 0)),
            pl.BlockSpec((D_FEAT, D_FEAT), lambda i: (0, 0)),
        ],
        out_specs=pl.BlockSpec((nb, D_FEAT), lambda i: (i, 0)),
        out_shape=jax.ShapeDtypeStruct((N_NODES, D_FEAT), f32),
    )(partials_a, partials_b, W_down)
    return out
